# baseline (speedup 1.0000x reference)
import functools

import jax
import jax.numpy as jnp
from jax import lax
from jax.experimental import pallas as pl
from jax.experimental.pallas import tpu as pltpu
from jax.experimental.pallas import tpu_sc as plsc

_N = 10000
_R = 16
_D = 128
_E = 320000
_W = 144
_NC = 2
_NS = 16
_EPT = _E // _NS
_NA = 10240
_RPT = _NA // _NS
_OCH = 128
_ZCH = 32
_CH = 64
_RING = 2 * _CH
_SL = 4000
_NSTRIP = _EPT // _SL
_CTRASH = _RING
_CCAP = _RING + 16


def _sc_segment_sums(xa, edges_packed):
    mesh = plsc.VectorSubcoreMesh(
        core_axis_name="c", subcore_axis_name="s",
        num_cores=_NC, num_subcores=_NS)
    zrows = jnp.zeros((_ZCH, _W), jnp.float32)

    @functools.partial(
        pl.kernel,
        out_type=jax.ShapeDtypeStruct((_R, _N, _W), jnp.float32),
        mesh=mesh,
        scratch_types=[
            pltpu.VMEM((2, _SL), jnp.int32),
            pltpu.VMEM((_CCAP,), jnp.int32),
            pltpu.VMEM((2, _CH), jnp.int32),
            pltpu.VMEM((2, _CH), jnp.int32),
            pltpu.VMEM((2, _CH, _W), jnp.float32),
            pltpu.VMEM((_ZCH, _W), jnp.float32),
            pltpu.VMEM_SHARED((_NA, _W), jnp.float32),
            pltpu.SemaphoreType.DMA,
            pltpu.SemaphoreType.DMA,
            pltpu.SemaphoreType.DMA,
            pltpu.SemaphoreType.DMA,
        ],
        compiler_params=pltpu.CompilerParams(
            needs_layout_passes=False, use_tc_tiling_on_sc=False),
    )
    def k(xa_hbm, ep_hbm, zr_hbm, out_hbm,
          strip, comp, src_idx, dst_idx, rows, zbuf, acc,
          sem_g, sem_s, sem2, sem_l):
        cid = lax.axis_index("c")
        tid = lax.axis_index("s")
        e0 = tid * _EPT
        pltpu.sync_copy(zr_hbm, zbuf)
        r0 = tid * _RPT
        for z in range(_RPT // _ZCH):
            pltpu.async_copy(zbuf, acc.at[pl.ds(r0 + z * _ZCH, _ZCH)], sem2)
        for z in range(_RPT // _ZCH):
            pltpu.make_async_copy(zbuf, acc.at[pl.ds(r0 + z * _ZCH, _ZCH)],
                                  sem2).wait()
        plsc.subcore_barrier()

        iota16 = lax.iota(jnp.int32, 16)
        dummy16 = jnp.full((16,), _N + (_N << 14), jnp.int32)
        trash16 = jnp.full((16,), _CTRASH, jnp.int32)
        ringm16 = jnp.full((16,), _RING - 1, jnp.int32)
        m14 = jnp.full((16,), 16383, jnp.int32)
        sh14 = jnp.full((16,), 14, jnp.int32)
        sh28 = jnp.full((16,), 28, jnp.int32)

        def wait_gather(q):
            pltpu.make_async_copy(
                xa_hbm.at[src_idx.at[q]], rows.at[q], sem_g).wait()

        def fire_sadd(q):
            pltpu.async_copy(rows.at[q], acc.at[dst_idx.at[q]], sem_s,
                             add=True)

        def wait_sadd(q):
            pltpu.make_async_copy(rows.at[q], acc.at[dst_idx.at[q]],
                                  sem_s).wait()

        def event(c):
            q = c & 1

            @pl.when(c >= 1)
            def _():
                wait_gather(1 - q)
                fire_sadd(1 - q)

            @pl.when(c >= 2)
            def _():
                wait_sadd(q)

            for v in range(_CH // 16):
                pk = comp[pl.ds(q * _CH + v * 16, 16)]
                src_idx.at[q][pl.ds(v * 16, 16)] = pk & m14
                dst_idx.at[q][pl.ds(v * 16, 16)] = (
                    lax.shift_right_logical(pk, sh14) & m14)
            pltpu.async_copy(xa_hbm.at[src_idx.at[q]], rows.at[q], sem_g)

        def fire_strip_load(s):
            pltpu.async_copy(ep_hbm.at[pl.ds(e0 + s * _SL, _SL)],
                             strip.at[s % 2], sem_l)

        def wait_strip_load(s):
            pltpu.make_async_copy(ep_hbm.at[pl.ds(e0 + s * _SL, _SL)],
                                  strip.at[s % 2], sem_l).wait()

        for p in range(_R // _NC):
            rel = _NC * p + cid
            rel16 = jnp.full((16,), rel, jnp.int32)

            fire_strip_load(0)

            def strip_body(s, kc, rel16=rel16):
                sb = strip.at[s % 2]
                wait_strip_load(s)

                @pl.when(s + 1 < _NSTRIP)
                def _():
                    fire_strip_load(s + 1)

                def scan_half(pk, kc, rel16=rel16):
                    m = lax.shift_right_logical(pk, sh28) == rel16
                    mv = m.astype(jnp.int32)
                    incl = plsc.cumsum(mv)
                    tgt = jnp.where(
                        m,
                        (jnp.full((16,), kc, jnp.int32) + incl - mv) & ringm16,
                        trash16)
                    plsc.store_scatter(comp, [tgt], pk)
                    return kc + incl[15]

                def scan_body(i, kc, sb=sb):
                    kc0 = kc
                    for h in range(4):
                        kc = scan_half(sb[pl.ds(i * 64 + h * 16, 16)], kc)

                    @pl.when(kc // _CH > kc0 // _CH)
                    def _():
                        event(kc0 // _CH)

                    return kc

                kc = lax.fori_loop(0, _SL // 64, scan_body, kc)
                kc0 = kc
                for h in range(2):
                    kc = scan_half(sb[pl.ds((_SL // 64) * 64 + h * 16, 16)], kc)

                @pl.when(kc // _CH > kc0 // _CH)
                def _():
                    event(kc0 // _CH)

                return kc

            kc = lax.fori_loop(0, _NSTRIP, strip_body, jnp.int32(0))
            cs = kc // _CH
            ct = (kc + (_CH - 1)) // _CH
            for v in range(_CH // 16):
                tgt = (jnp.full((16,), kc + v * 16, jnp.int32) + iota16) \
                    & ringm16
                plsc.store_scatter(comp, [tgt], dummy16)

            @pl.when(ct > cs)
            def _():
                event(cs)

            @pl.when(ct >= 1)
            def _():
                wait_gather((ct - 1) & 1)
                fire_sadd((ct - 1) & 1)

            @pl.when(ct >= 2)
            def _():
                wait_sadd((ct - 2) & 1)

            @pl.when(ct >= 1)
            def _():
                wait_sadd((ct - 1) & 1)

            plsc.subcore_barrier()
            for z in range(3):
                sl = pl.ds(r0 + z * _OCH, _OCH)
                pltpu.async_copy(acc.at[sl], out_hbm.at[rel].at[sl], sem2)
            for z in range(3):
                sl = pl.ds(r0 + z * _OCH, _OCH)
                pltpu.make_async_copy(acc.at[sl], out_hbm.at[rel].at[sl],
                                      sem2).wait()

            @pl.when(tid < _NS - 1)
            def _():
                for z in range(3, _RPT // _OCH):
                    sl = pl.ds(r0 + z * _OCH, _OCH)
                    pltpu.async_copy(acc.at[sl], out_hbm.at[rel].at[sl], sem2)
                for z in range(3, _RPT // _OCH):
                    sl = pl.ds(r0 + z * _OCH, _OCH)
                    pltpu.make_async_copy(acc.at[sl], out_hbm.at[rel].at[sl],
                                          sem2).wait()

            @pl.when(tid == _NS - 1)
            def _():
                sl = pl.ds(_N - 16, 16)
                pltpu.sync_copy(acc.at[sl], out_hbm.at[rel].at[sl])

            for z in range(_RPT // _ZCH):
                sl = pl.ds(r0 + z * _ZCH, _ZCH)
                pltpu.async_copy(zbuf, acc.at[sl], sem2)
            for z in range(_RPT // _ZCH):
                sl = pl.ds(r0 + z * _ZCH, _ZCH)
                pltpu.make_async_copy(zbuf, acc.at[sl], sem2).wait()
            plsc.subcore_barrier()

    return k(xa, edges_packed, zrows)


def _tc_body(sums_ref, x_ref, wd_ref, root_ref, b_ref, g_ref, bb_ref, out_ref,
             *, residual):
    xb = x_ref[...]
    acc = jnp.dot(xb, root_ref[...], preferred_element_type=jnp.float32)
    acc = acc + b_ref[...]
    for r in range(_R):
        sr = sums_ref[r]
        cnt = jnp.maximum(sr[:, 128:129], 1.0)
        mean = sr[:, :128] / cnt
        acc = acc + jnp.dot(mean, wd_ref[r], preferred_element_type=jnp.float32)
    if residual:
        acc = acc + xb
    mu = jnp.mean(acc, axis=-1, keepdims=True)
    var = jnp.mean((acc - mu) ** 2, axis=-1, keepdims=True)
    y = (acc - mu) * lax.rsqrt(var + 1e-5) * g_ref[...] + bb_ref[...]
    out_ref[...] = jnp.maximum(y, 0.0)


def _tc_layer(sums, x, wd, root, bias, g, bb, *, residual):
    nb = 1000
    grid = (_N // nb,)
    return pl.pallas_call(
        functools.partial(_tc_body, residual=residual),
        grid=grid,
        in_specs=[
            pl.BlockSpec((_R, nb, _W), lambda i: (0, i, 0)),
            pl.BlockSpec((nb, _D), lambda i: (i, 0)),
            pl.BlockSpec((_R, _D, _D), lambda i: (0, 0, 0)),
            pl.BlockSpec((_D, _D), lambda i: (0, 0)),
            pl.BlockSpec((1, _D), lambda i: (0, 0)),
            pl.BlockSpec((1, _D), lambda i: (0, 0)),
            pl.BlockSpec((1, _D), lambda i: (0, 0)),
        ],
        out_specs=pl.BlockSpec((nb, _D), lambda i: (i, 0)),
        out_shape=jax.ShapeDtypeStruct((_N, _D), jnp.float32),
    )(sums, x, wd, root, bias, g, bb)


def _expand_blockdiag(w):
    return jax.vmap(lambda wr: jax.scipy.linalg.block_diag(*[wr[b] for b in range(4)]))(w)


def _augment(x):
    xa = jnp.zeros((_N + 1, _W), jnp.float32)
    xa = xa.at[:_N, :_D].set(x)
    xa = xa.at[:_N, _D].set(1.0)
    return xa


def kernel(edge_index, edge_type, entity_emb, w0, root0, b0, ln_g0, ln_b0,
           w1, root1, b1, ln_g1, ln_b1):
    x = entity_emb
    edges_packed = ((edge_type << 28) | (edge_index[1] << 14) | edge_index[0])
    wd0 = _expand_blockdiag(w0)
    wd1 = _expand_blockdiag(w1)
    sums0 = _sc_segment_sums(_augment(x), edges_packed)
    x1 = _tc_layer(sums0, x, wd0, root0, b0.reshape(1, -1),
                   ln_g0.reshape(1, -1), ln_b0.reshape(1, -1), residual=False)
    sums1 = _sc_segment_sums(_augment(x1), edges_packed)
    x2 = _tc_layer(sums1, x1, wd1, root1, b1.reshape(1, -1),
                   ln_g1.reshape(1, -1), ln_b1.reshape(1, -1), residual=True)
    return x2


# speedup vs baseline: 1.1360x; 1.1360x over previous
import functools

import jax
import jax.numpy as jnp
from jax import lax
from jax.experimental import pallas as pl
from jax.experimental.pallas import tpu as pltpu
from jax.experimental.pallas import tpu_sc as plsc

_N = 10000
_R = 16
_D = 128
_E = 320000
_W = 144
_NC = 2
_NS = 16
_EPT = _E // _NS
_NA = 10240
_RPT = _NA // _NS
_OCH = 128
_ZCH = 32
_CH = 32
_NSLOT = 4
_RING = _NSLOT * _CH
_SL = 4000
_NSTRIP = _EPT // _SL
_CTRASH = _RING
_CCAP = _RING + 16


def _sc_segment_sums(xa, edges_packed):
    mesh = plsc.VectorSubcoreMesh(
        core_axis_name="c", subcore_axis_name="s",
        num_cores=_NC, num_subcores=_NS)
    zrows = jnp.zeros((_ZCH, _W), jnp.float32)

    @functools.partial(
        pl.kernel,
        out_type=jax.ShapeDtypeStruct((_R, _N, _W), jnp.float32),
        mesh=mesh,
        scratch_types=[
            pltpu.VMEM((2, _SL), jnp.int32),
            pltpu.VMEM((_CCAP,), jnp.int32),
            pltpu.VMEM((_NSLOT, _CH), jnp.int32),
            pltpu.VMEM((_NSLOT, _CH), jnp.int32),
            pltpu.VMEM((_NSLOT, _CH, _W), jnp.float32),
            pltpu.VMEM((_ZCH, _W), jnp.float32),
            pltpu.VMEM_SHARED((_NA, _W), jnp.float32),
            pltpu.SemaphoreType.DMA,
            pltpu.SemaphoreType.DMA,
            pltpu.SemaphoreType.DMA,
            pltpu.SemaphoreType.DMA,
        ],
        compiler_params=pltpu.CompilerParams(
            needs_layout_passes=False, use_tc_tiling_on_sc=False),
    )
    def k(xa_hbm, ep_hbm, zr_hbm, out_hbm,
          strip, comp, src_idx, dst_idx, rows, zbuf, acc,
          sem_g, sem_s, sem2, sem_l):
        cid = lax.axis_index("c")
        tid = lax.axis_index("s")
        e0 = tid * _EPT
        pltpu.sync_copy(zr_hbm, zbuf)
        r0 = tid * _RPT
        for z in range(_RPT // _ZCH):
            pltpu.async_copy(zbuf, acc.at[pl.ds(r0 + z * _ZCH, _ZCH)], sem2)
        for z in range(_RPT // _ZCH):
            pltpu.make_async_copy(zbuf, acc.at[pl.ds(r0 + z * _ZCH, _ZCH)],
                                  sem2).wait()
        plsc.subcore_barrier()

        iota16 = lax.iota(jnp.int32, 16)
        dummy16 = jnp.full((16,), _N + (_N << 14), jnp.int32)
        trash16 = jnp.full((16,), _CTRASH, jnp.int32)
        ringm16 = jnp.full((16,), _RING - 1, jnp.int32)
        m14 = jnp.full((16,), 16383, jnp.int32)
        sh14 = jnp.full((16,), 14, jnp.int32)
        sh28 = jnp.full((16,), 28, jnp.int32)

        def wait_gather(q):
            pltpu.make_async_copy(
                xa_hbm.at[src_idx.at[q]], rows.at[q], sem_g).wait()

        def fire_sadd(q):
            pltpu.async_copy(rows.at[q], acc.at[dst_idx.at[q]], sem_s,
                             add=True)

        def wait_sadd(q):
            pltpu.make_async_copy(rows.at[q], acc.at[dst_idx.at[q]],
                                  sem_s).wait()

        def event(c):
            q = c & (_NSLOT - 1)

            @pl.when(c >= _NSLOT)
            def _():
                wait_sadd(q)

            for v in range(_CH // 16):
                pk = comp[pl.ds(q * _CH + v * 16, 16)]
                src_idx.at[q][pl.ds(v * 16, 16)] = pk & m14
                dst_idx.at[q][pl.ds(v * 16, 16)] = (
                    lax.shift_right_logical(pk, sh14) & m14)
            pltpu.async_copy(xa_hbm.at[src_idx.at[q]], rows.at[q], sem_g)

            @pl.when(c >= _NSLOT - 1)
            def _():
                qr = (c - (_NSLOT - 1)) & (_NSLOT - 1)
                wait_gather(qr)
                fire_sadd(qr)

        def fire_strip_load(s):
            pltpu.async_copy(ep_hbm.at[pl.ds(e0 + s * _SL, _SL)],
                             strip.at[s % 2], sem_l)

        def wait_strip_load(s):
            pltpu.make_async_copy(ep_hbm.at[pl.ds(e0 + s * _SL, _SL)],
                                  strip.at[s % 2], sem_l).wait()

        for p in range(_R // _NC):
            rel = _NC * p + cid
            rel16 = jnp.full((16,), rel, jnp.int32)

            fire_strip_load(0)

            def strip_body(s, kc, rel16=rel16):
                sb = strip.at[s % 2]
                wait_strip_load(s)

                @pl.when(s + 1 < _NSTRIP)
                def _():
                    fire_strip_load(s + 1)

                def scan_half(pk, kc, rel16=rel16):
                    m = lax.shift_right_logical(pk, sh28) == rel16
                    mv = m.astype(jnp.int32)
                    incl = plsc.cumsum(mv)
                    tgt = jnp.where(
                        m,
                        (jnp.full((16,), kc, jnp.int32) + incl - mv) & ringm16,
                        trash16)
                    plsc.store_scatter(comp, [tgt], pk)
                    return kc + incl[15]

                def scan_body(i, kc, sb=sb):
                    kc0 = kc
                    for h in range(2):
                        kc = scan_half(sb[pl.ds(i * 32 + h * 16, 16)], kc)

                    @pl.when(kc // _CH > kc0 // _CH)
                    def _():
                        event(kc0 // _CH)

                    return kc

                return lax.fori_loop(0, _SL // 32, scan_body, kc)

            kc = lax.fori_loop(0, _NSTRIP, strip_body, jnp.int32(0))
            cs = kc // _CH
            ct = (kc + (_CH - 1)) // _CH
            for v in range(_CH // 16):
                tgt = (jnp.full((16,), kc + v * 16, jnp.int32) + iota16) \
                    & ringm16
                plsc.store_scatter(comp, [tgt], dummy16)

            @pl.when(ct > cs)
            def _():
                event(cs)

            for d in range(_NSLOT - 1, 0, -1):
                @pl.when(ct >= d)
                def _(d=d):
                    qr = (ct - d) & (_NSLOT - 1)
                    wait_gather(qr)
                    fire_sadd(qr)

            for d in range(_NSLOT, 0, -1):
                @pl.when(ct >= d)
                def _(d=d):
                    wait_sadd((ct - d) & (_NSLOT - 1))

            plsc.subcore_barrier()
            for z in range(3):
                sl = pl.ds(r0 + z * _OCH, _OCH)
                pltpu.async_copy(acc.at[sl], out_hbm.at[rel].at[sl], sem2)
            for z in range(3):
                sl = pl.ds(r0 + z * _OCH, _OCH)
                pltpu.make_async_copy(acc.at[sl], out_hbm.at[rel].at[sl],
                                      sem2).wait()

            @pl.when(tid < _NS - 1)
            def _():
                for z in range(3, _RPT // _OCH):
                    sl = pl.ds(r0 + z * _OCH, _OCH)
                    pltpu.async_copy(acc.at[sl], out_hbm.at[rel].at[sl], sem2)
                for z in range(3, _RPT // _OCH):
                    sl = pl.ds(r0 + z * _OCH, _OCH)
                    pltpu.make_async_copy(acc.at[sl], out_hbm.at[rel].at[sl],
                                          sem2).wait()

            @pl.when(tid == _NS - 1)
            def _():
                sl = pl.ds(_N - 16, 16)
                pltpu.sync_copy(acc.at[sl], out_hbm.at[rel].at[sl])

            for z in range(_RPT // _ZCH):
                sl = pl.ds(r0 + z * _ZCH, _ZCH)
                pltpu.async_copy(zbuf, acc.at[sl], sem2)
            for z in range(_RPT // _ZCH):
                sl = pl.ds(r0 + z * _ZCH, _ZCH)
                pltpu.make_async_copy(zbuf, acc.at[sl], sem2).wait()
            plsc.subcore_barrier()

    return k(xa, edges_packed, zrows)


def _tc_body(sums_ref, x_ref, wd_ref, root_ref, b_ref, g_ref, bb_ref, out_ref,
             *, residual):
    xb = x_ref[...]
    acc = jnp.dot(xb, root_ref[...], preferred_element_type=jnp.float32)
    acc = acc + b_ref[...]
    for r in range(_R):
        sr = sums_ref[r]
        cnt = jnp.maximum(sr[:, 128:129], 1.0)
        mean = sr[:, :128] / cnt
        acc = acc + jnp.dot(mean, wd_ref[r], preferred_element_type=jnp.float32)
    if residual:
        acc = acc + xb
    mu = jnp.mean(acc, axis=-1, keepdims=True)
    var = jnp.mean((acc - mu) ** 2, axis=-1, keepdims=True)
    y = (acc - mu) * lax.rsqrt(var + 1e-5) * g_ref[...] + bb_ref[...]
    out_ref[...] = jnp.maximum(y, 0.0)


def _tc_layer(sums, x, wd, root, bias, g, bb, *, residual):
    nb = 1000
    grid = (_N // nb,)
    return pl.pallas_call(
        functools.partial(_tc_body, residual=residual),
        grid=grid,
        in_specs=[
            pl.BlockSpec((_R, nb, _W), lambda i: (0, i, 0)),
            pl.BlockSpec((nb, _D), lambda i: (i, 0)),
            pl.BlockSpec((_R, _D, _D), lambda i: (0, 0, 0)),
            pl.BlockSpec((_D, _D), lambda i: (0, 0)),
            pl.BlockSpec((1, _D), lambda i: (0, 0)),
            pl.BlockSpec((1, _D), lambda i: (0, 0)),
            pl.BlockSpec((1, _D), lambda i: (0, 0)),
        ],
        out_specs=pl.BlockSpec((nb, _D), lambda i: (i, 0)),
        out_shape=jax.ShapeDtypeStruct((_N, _D), jnp.float32),
    )(sums, x, wd, root, bias, g, bb)


def _expand_blockdiag(w):
    return jax.vmap(lambda wr: jax.scipy.linalg.block_diag(*[wr[b] for b in range(4)]))(w)


def _augment(x):
    xa = jnp.zeros((_N + 1, _W), jnp.float32)
    xa = xa.at[:_N, :_D].set(x)
    xa = xa.at[:_N, _D].set(1.0)
    return xa


def kernel(edge_index, edge_type, entity_emb, w0, root0, b0, ln_g0, ln_b0,
           w1, root1, b1, ln_g1, ln_b1):
    x = entity_emb
    edges_packed = ((edge_type << 28) | (edge_index[1] << 14) | edge_index[0])
    wd0 = _expand_blockdiag(w0)
    wd1 = _expand_blockdiag(w1)
    sums0 = _sc_segment_sums(_augment(x), edges_packed)
    x1 = _tc_layer(sums0, x, wd0, root0, b0.reshape(1, -1),
                   ln_g0.reshape(1, -1), ln_b0.reshape(1, -1), residual=False)
    sums1 = _sc_segment_sums(_augment(x1), edges_packed)
    x2 = _tc_layer(sums1, x1, wd1, root1, b1.reshape(1, -1),
                   ln_g1.reshape(1, -1), ln_b1.reshape(1, -1), residual=True)
    return x2


# speedup vs baseline: 1.2353x; 1.0874x over previous
import functools

import jax
import jax.numpy as jnp
from jax import lax
from jax.experimental import pallas as pl
from jax.experimental.pallas import tpu as pltpu
from jax.experimental.pallas import tpu_sc as plsc

_N = 10000
_R = 16
_D = 128
_E = 320000
_W = 144
_NC = 2
_NS = 16
_EPT = _E // _NS
_NA = 10240
_RPT = _NA // _NS
_OCH = 128
_ZCH = 32
_CH = 32
_NSLOT = 4
_RING = _NSLOT * _CH
_SL = 4000
_NSTRIP = _EPT // _SL
_CTRASH = _RING
_CCAP = _RING + 16
_BCAP = (_EPT // _CH + 1) * _CH


def _sc_segment_sums(xa, edges_packed):
    mesh = plsc.VectorSubcoreMesh(
        core_axis_name="c", subcore_axis_name="s",
        num_cores=_NC, num_subcores=_NS)
    zrows = jnp.zeros((_ZCH, _W), jnp.float32)

    @functools.partial(
        pl.kernel,
        out_type=(
            jax.ShapeDtypeStruct((_R, _N, _W), jnp.float32),
            jax.ShapeDtypeStruct((_NC, _NS, _R // _NC, _BCAP), jnp.int32),
            jax.ShapeDtypeStruct((_NC, _NS, _R // _NC), jnp.int32),
        ),
        mesh=mesh,
        scratch_types=[
            pltpu.VMEM((2, _SL), jnp.int32),
            pltpu.VMEM((_CCAP,), jnp.int32),
            pltpu.VMEM((_NSLOT, _CH), jnp.int32),
            pltpu.VMEM((_NSLOT, _CH), jnp.int32),
            pltpu.VMEM((_NSLOT, _CH, _W), jnp.float32),
            pltpu.VMEM((_ZCH, _W), jnp.float32),
            pltpu.VMEM((16,), jnp.int32),
            pltpu.VMEM_SHARED((_NA, _W), jnp.float32),
            pltpu.SemaphoreType.DMA,
            pltpu.SemaphoreType.DMA,
            pltpu.SemaphoreType.DMA,
            pltpu.SemaphoreType.DMA,
            pltpu.SemaphoreType.DMA,
        ],
        compiler_params=pltpu.CompilerParams(
            needs_layout_passes=False, use_tc_tiling_on_sc=False),
    )
    def k(xa_hbm, ep_hbm, zr_hbm, out_hbm, bk_hbm, cnt_hbm,
          strip, comp, src_idx, dst_idx, rows, zbuf, kcs, acc,
          sem_g, sem_s, sem2, sem_l, sem_b):
        cid = lax.axis_index("c")
        tid = lax.axis_index("s")
        e0 = tid * _EPT
        pltpu.sync_copy(zr_hbm, zbuf)
        r0 = tid * _RPT
        for z in range(_RPT // _ZCH):
            pltpu.async_copy(zbuf, acc.at[pl.ds(r0 + z * _ZCH, _ZCH)], sem2)
        for z in range(_RPT // _ZCH):
            pltpu.make_async_copy(zbuf, acc.at[pl.ds(r0 + z * _ZCH, _ZCH)],
                                  sem2).wait()
        plsc.subcore_barrier()

        iota16 = lax.iota(jnp.int32, 16)
        dummy16 = jnp.full((16,), _N + (_N << 14), jnp.int32)
        trash16 = jnp.full((16,), _CTRASH, jnp.int32)
        ringm16 = jnp.full((16,), _RING - 1, jnp.int32)
        m14 = jnp.full((16,), 16383, jnp.int32)
        sh14 = jnp.full((16,), 14, jnp.int32)
        sh28 = jnp.full((16,), 28, jnp.int32)

        def wait_gather(q):
            pltpu.make_async_copy(
                xa_hbm.at[src_idx.at[q]], rows.at[q], sem_g).wait()

        def fire_sadd(q):
            pltpu.async_copy(rows.at[q], acc.at[dst_idx.at[q]], sem_s,
                             add=True)

        def wait_sadd(q):
            pltpu.make_async_copy(rows.at[q], acc.at[dst_idx.at[q]],
                                  sem_s).wait()

        def fire_spill(c, p):
            q = c & (_NSLOT - 1)
            pltpu.async_copy(
                comp.at[pl.ds(q * _CH, _CH)],
                bk_hbm.at[cid, tid, p, pl.ds(c * _CH, _CH)], sem_b)

        def wait_spill(c, p):
            q = c & (_NSLOT - 1)
            pltpu.make_async_copy(
                comp.at[pl.ds(q * _CH, _CH)],
                bk_hbm.at[cid, tid, p, pl.ds(c * _CH, _CH)], sem_b).wait()

        def event(c, p):
            q = c & (_NSLOT - 1)

            @pl.when(c >= _NSLOT)
            def _():
                wait_sadd(q)

            for v in range(_CH // 16):
                pk = comp[pl.ds(q * _CH + v * 16, 16)]
                src_idx.at[q][pl.ds(v * 16, 16)] = pk & m14
                dst_idx.at[q][pl.ds(v * 16, 16)] = (
                    lax.shift_right_logical(pk, sh14) & m14)
            pltpu.async_copy(xa_hbm.at[src_idx.at[q]], rows.at[q], sem_g)
            fire_spill(c, p)

            @pl.when(c >= _NSLOT - 1)
            def _():
                qr = (c - (_NSLOT - 1)) & (_NSLOT - 1)
                wait_gather(qr)
                fire_sadd(qr)

            @pl.when(c >= 3)
            def _():
                wait_spill(c - 3, p)

        def fire_strip_load(s):
            pltpu.async_copy(ep_hbm.at[pl.ds(e0 + s * _SL, _SL)],
                             strip.at[s % 2], sem_l)

        def wait_strip_load(s):
            pltpu.make_async_copy(ep_hbm.at[pl.ds(e0 + s * _SL, _SL)],
                                  strip.at[s % 2], sem_l).wait()

        kcs_vec = jnp.zeros((16,), jnp.int32)
        for p in range(_R // _NC):
            rel = _NC * p + cid
            rel16 = jnp.full((16,), rel, jnp.int32)

            fire_strip_load(0)

            def strip_body(s, kc, rel16=rel16):
                sb = strip.at[s % 2]
                wait_strip_load(s)

                @pl.when(s + 1 < _NSTRIP)
                def _():
                    fire_strip_load(s + 1)

                def scan_half(pk, kc, rel16=rel16):
                    m = lax.shift_right_logical(pk, sh28) == rel16
                    mv = m.astype(jnp.int32)
                    incl = plsc.cumsum(mv)
                    tgt = jnp.where(
                        m,
                        (jnp.full((16,), kc, jnp.int32) + incl - mv) & ringm16,
                        trash16)
                    plsc.store_scatter(comp, [tgt], pk)
                    return kc + incl[15]

                def scan_body(i, kc, sb=sb, p=p):
                    kc0 = kc
                    for h in range(2):
                        kc = scan_half(sb[pl.ds(i * 32 + h * 16, 16)], kc)

                    @pl.when(kc // _CH > kc0 // _CH)
                    def _():
                        event(kc0 // _CH, p)

                    return kc

                return lax.fori_loop(0, _SL // 32, scan_body, kc)

            kc = lax.fori_loop(0, _NSTRIP, strip_body, jnp.int32(0))
            cs = kc // _CH
            ct = (kc + (_CH - 1)) // _CH
            for v in range(_CH // 16):
                tgt = (jnp.full((16,), kc + v * 16, jnp.int32) + iota16) \
                    & ringm16
                plsc.store_scatter(comp, [tgt], dummy16)

            @pl.when(ct > cs)
            def _():
                event(cs, p)

            for d in range(_NSLOT - 1, 0, -1):
                @pl.when(ct >= d)
                def _(d=d):
                    qr = (ct - d) & (_NSLOT - 1)
                    wait_gather(qr)
                    fire_sadd(qr)

            for d in range(_NSLOT, 0, -1):
                @pl.when(ct >= d)
                def _(d=d):
                    wait_sadd((ct - d) & (_NSLOT - 1))

            for d in range(3, 0, -1):
                @pl.when(ct >= d)
                def _(d=d):
                    wait_spill(ct - d, p)

            kcs_vec = jnp.where(iota16 == jnp.full((16,), p, jnp.int32),
                                jnp.full((16,), ct, jnp.int32), kcs_vec)

            plsc.subcore_barrier()
            for z in range(3):
                sl = pl.ds(r0 + z * _OCH, _OCH)
                pltpu.async_copy(acc.at[sl], out_hbm.at[rel].at[sl], sem2)
            for z in range(3):
                sl = pl.ds(r0 + z * _OCH, _OCH)
                pltpu.make_async_copy(acc.at[sl], out_hbm.at[rel].at[sl],
                                      sem2).wait()

            @pl.when(tid < _NS - 1)
            def _():
                for z in range(3, _RPT // _OCH):
                    sl = pl.ds(r0 + z * _OCH, _OCH)
                    pltpu.async_copy(acc.at[sl], out_hbm.at[rel].at[sl], sem2)
                for z in range(3, _RPT // _OCH):
                    sl = pl.ds(r0 + z * _OCH, _OCH)
                    pltpu.make_async_copy(acc.at[sl], out_hbm.at[rel].at[sl],
                                          sem2).wait()

            @pl.when(tid == _NS - 1)
            def _():
                sl = pl.ds(_N - 16, 16)
                pltpu.sync_copy(acc.at[sl], out_hbm.at[rel].at[sl])

            for z in range(_RPT // _ZCH):
                sl = pl.ds(r0 + z * _ZCH, _ZCH)
                pltpu.async_copy(zbuf, acc.at[sl], sem2)
            for z in range(_RPT // _ZCH):
                sl = pl.ds(r0 + z * _ZCH, _ZCH)
                pltpu.make_async_copy(zbuf, acc.at[sl], sem2).wait()
            plsc.subcore_barrier()

        kcs[pl.ds(0, 16)] = kcs_vec
        pltpu.sync_copy(kcs.at[pl.ds(0, _R // _NC)], cnt_hbm.at[cid, tid])

    return k(xa, edges_packed, zrows)


def _sc_segment_sums_from_buckets(xa, buckets, counts):
    mesh = plsc.VectorSubcoreMesh(
        core_axis_name="c", subcore_axis_name="s",
        num_cores=_NC, num_subcores=_NS)
    zrows = jnp.zeros((_ZCH, _W), jnp.float32)

    @functools.partial(
        pl.kernel,
        out_type=jax.ShapeDtypeStruct((_R, _N, _W), jnp.float32),
        mesh=mesh,
        scratch_types=[
            pltpu.VMEM((2, _CH), jnp.int32),
            pltpu.VMEM((_NSLOT, _CH), jnp.int32),
            pltpu.VMEM((_NSLOT, _CH), jnp.int32),
            pltpu.VMEM((_NSLOT, _CH, _W), jnp.float32),
            pltpu.VMEM((_ZCH, _W), jnp.float32),
            pltpu.VMEM((16,), jnp.int32),
            pltpu.VMEM_SHARED((_NA, _W), jnp.float32),
            pltpu.SemaphoreType.DMA,
            pltpu.SemaphoreType.DMA,
            pltpu.SemaphoreType.DMA,
            pltpu.SemaphoreType.DMA,
        ],
        compiler_params=pltpu.CompilerParams(
            needs_layout_passes=False, use_tc_tiling_on_sc=False),
    )
    def k(xa_hbm, bk_hbm, cnt_hbm, zr_hbm, out_hbm,
          bl, src_idx, dst_idx, rows, zbuf, cnt_s, acc,
          sem_g, sem_s, sem2, sem_b):
        cid = lax.axis_index("c")
        tid = lax.axis_index("s")
        pltpu.sync_copy(zr_hbm, zbuf)
        pltpu.sync_copy(cnt_hbm.at[cid, tid], cnt_s.at[pl.ds(0, _R // _NC)])
        cnt_v = cnt_s[pl.ds(0, 16)]
        r0 = tid * _RPT
        for z in range(_RPT // _ZCH):
            pltpu.async_copy(zbuf, acc.at[pl.ds(r0 + z * _ZCH, _ZCH)], sem2)
        for z in range(_RPT // _ZCH):
            pltpu.make_async_copy(zbuf, acc.at[pl.ds(r0 + z * _ZCH, _ZCH)],
                                  sem2).wait()
        plsc.subcore_barrier()

        m14 = jnp.full((16,), 16383, jnp.int32)
        sh14 = jnp.full((16,), 14, jnp.int32)

        def wait_gather(q):
            pltpu.make_async_copy(
                xa_hbm.at[src_idx.at[q]], rows.at[q], sem_g).wait()

        def fire_sadd(q):
            pltpu.async_copy(rows.at[q], acc.at[dst_idx.at[q]], sem_s,
                             add=True)

        def wait_sadd(q):
            pltpu.make_async_copy(rows.at[q], acc.at[dst_idx.at[q]],
                                  sem_s).wait()

        def fire_bload(c, p):
            pltpu.async_copy(bk_hbm.at[cid, tid, p, pl.ds(c * _CH, _CH)],
                             bl.at[c & 1], sem_b)

        def wait_bload(c, p):
            pltpu.make_async_copy(bk_hbm.at[cid, tid, p, pl.ds(c * _CH, _CH)],
                                  bl.at[c & 1], sem_b).wait()

        for p in range(_R // _NC):
            rel = _NC * p + cid
            ct = cnt_v[p]

            @pl.when(ct >= 1)
            def _(p=p):
                fire_bload(0, p)

            def chunk_body(c, _, p=p):
                q = c & (_NSLOT - 1)

                @pl.when(c + 1 < ct)
                def _():
                    fire_bload(c + 1, p)

                wait_bload(c, p)

                @pl.when(c >= _NSLOT)
                def _():
                    wait_sadd(q)

                for v in range(_CH // 16):
                    pk = bl.at[c & 1][pl.ds(v * 16, 16)]
                    src_idx.at[q][pl.ds(v * 16, 16)] = pk & m14
                    dst_idx.at[q][pl.ds(v * 16, 16)] = (
                        lax.shift_right_logical(pk, sh14) & m14)
                pltpu.async_copy(xa_hbm.at[src_idx.at[q]], rows.at[q], sem_g)

                @pl.when(c >= _NSLOT - 1)
                def _():
                    qr = (c - (_NSLOT - 1)) & (_NSLOT - 1)
                    wait_gather(qr)
                    fire_sadd(qr)

                return 0

            lax.fori_loop(0, ct, chunk_body, 0)
            for d in range(_NSLOT - 1, 0, -1):
                @pl.when(ct >= d)
                def _(d=d):
                    qr = (ct - d) & (_NSLOT - 1)
                    wait_gather(qr)
                    fire_sadd(qr)

            for d in range(_NSLOT, 0, -1):
                @pl.when(ct >= d)
                def _(d=d):
                    wait_sadd((ct - d) & (_NSLOT - 1))

            plsc.subcore_barrier()
            for z in range(3):
                sl = pl.ds(r0 + z * _OCH, _OCH)
                pltpu.async_copy(acc.at[sl], out_hbm.at[rel].at[sl], sem2)
            for z in range(3):
                sl = pl.ds(r0 + z * _OCH, _OCH)
                pltpu.make_async_copy(acc.at[sl], out_hbm.at[rel].at[sl],
                                      sem2).wait()

            @pl.when(tid < _NS - 1)
            def _(rel=rel):
                for z in range(3, _RPT // _OCH):
                    sl = pl.ds(r0 + z * _OCH, _OCH)
                    pltpu.async_copy(acc.at[sl], out_hbm.at[rel].at[sl], sem2)
                for z in range(3, _RPT // _OCH):
                    sl = pl.ds(r0 + z * _OCH, _OCH)
                    pltpu.make_async_copy(acc.at[sl], out_hbm.at[rel].at[sl],
                                          sem2).wait()

            @pl.when(tid == _NS - 1)
            def _(rel=rel):
                sl = pl.ds(_N - 16, 16)
                pltpu.sync_copy(acc.at[sl], out_hbm.at[rel].at[sl])

            for z in range(_RPT // _ZCH):
                sl = pl.ds(r0 + z * _ZCH, _ZCH)
                pltpu.async_copy(zbuf, acc.at[sl], sem2)
            for z in range(_RPT // _ZCH):
                sl = pl.ds(r0 + z * _ZCH, _ZCH)
                pltpu.make_async_copy(zbuf, acc.at[sl], sem2).wait()
            plsc.subcore_barrier()

    return k(xa, buckets, counts, zrows)


def _tc_body(sums_ref, x_ref, wd_ref, root_ref, b_ref, g_ref, bb_ref, out_ref,
             *, residual):
    xb = x_ref[...]
    acc = jnp.dot(xb, root_ref[...], preferred_element_type=jnp.float32)
    acc = acc + b_ref[...]
    for r in range(_R):
        sr = sums_ref[r]
        cnt = jnp.maximum(sr[:, 128:129], 1.0)
        mean = sr[:, :128] / cnt
        acc = acc + jnp.dot(mean, wd_ref[r], preferred_element_type=jnp.float32)
    if residual:
        acc = acc + xb
    mu = jnp.mean(acc, axis=-1, keepdims=True)
    var = jnp.mean((acc - mu) ** 2, axis=-1, keepdims=True)
    y = (acc - mu) * lax.rsqrt(var + 1e-5) * g_ref[...] + bb_ref[...]
    out_ref[...] = jnp.maximum(y, 0.0)


def _tc_layer(sums, x, wd, root, bias, g, bb, *, residual):
    nb = 1000
    grid = (_N // nb,)
    return pl.pallas_call(
        functools.partial(_tc_body, residual=residual),
        grid=grid,
        in_specs=[
            pl.BlockSpec((_R, nb, _W), lambda i: (0, i, 0)),
            pl.BlockSpec((nb, _D), lambda i: (i, 0)),
            pl.BlockSpec((_R, _D, _D), lambda i: (0, 0, 0)),
            pl.BlockSpec((_D, _D), lambda i: (0, 0)),
            pl.BlockSpec((1, _D), lambda i: (0, 0)),
            pl.BlockSpec((1, _D), lambda i: (0, 0)),
            pl.BlockSpec((1, _D), lambda i: (0, 0)),
        ],
        out_specs=pl.BlockSpec((nb, _D), lambda i: (i, 0)),
        out_shape=jax.ShapeDtypeStruct((_N, _D), jnp.float32),
    )(sums, x, wd, root, bias, g, bb)


def _expand_blockdiag(w):
    return jax.vmap(lambda wr: jax.scipy.linalg.block_diag(*[wr[b] for b in range(4)]))(w)


def _augment(x):
    xa = jnp.zeros((_N + 1, _W), jnp.float32)
    xa = xa.at[:_N, :_D].set(x)
    xa = xa.at[:_N, _D].set(1.0)
    return xa


def kernel(edge_index, edge_type, entity_emb, w0, root0, b0, ln_g0, ln_b0,
           w1, root1, b1, ln_g1, ln_b1):
    x = entity_emb
    edges_packed = ((edge_type << 28) | (edge_index[1] << 14) | edge_index[0])
    wd0 = _expand_blockdiag(w0)
    wd1 = _expand_blockdiag(w1)
    sums0, buckets, counts = _sc_segment_sums(_augment(x), edges_packed)
    x1 = _tc_layer(sums0, x, wd0, root0, b0.reshape(1, -1),
                   ln_g0.reshape(1, -1), ln_b0.reshape(1, -1), residual=False)
    sums1 = _sc_segment_sums_from_buckets(_augment(x1), buckets, counts)
    x2 = _tc_layer(sums1, x1, wd1, root1, b1.reshape(1, -1),
                   ln_g1.reshape(1, -1), ln_b1.reshape(1, -1), residual=True)
    return x2
